# Initial kernel scaffold; baseline (speedup 1.0000x reference)
import functools

import jax
import jax.numpy as jnp
from jax import lax
from jax.experimental import pallas as pl
from jax.experimental.pallas import tpu as pltpu
from jax.experimental.pallas import tpu_sc as plsc

_N = 10000
_T = 4
_H = 128
_E = 320000

_NS = 16
_EPT = _E // _NS
_CHUNK = 80
_NCHUNK = _EPT // _CHUNK
_LANES = 16
_RPT = _N // _NS


def _sc_agg_body(with_counts, x_hbm, src_hbm, dst_hbm, *refs):
  if with_counts:
    agg_hbm, cnt_hbm = refs[0], refs[1]
    refs = refs[2:]
  else:
    agg_hbm = refs[0]
    cnt_hbm = None
    refs = refs[1:]
  (src_v, gidx_v, dst2_v, rows_v, ones_v, zrow_v, zcnt_v, acc_s, cnt_s,
   sem) = refs

  c = lax.axis_index("c")
  s = lax.axis_index("s")
  ebase = s * _EPT
  pltpu.sync_copy(src_hbm.at[pl.ds(ebase, _EPT)], src_v)

  zf = jnp.zeros((_LANES,), jnp.float32)

  pltpu.sync_copy(dst_hbm.at[pl.ds(ebase, _EPT)], gidx_v)

  def dst_fmt(k, carry):
    for j in range(_CHUNK // _LANES):
      dst2_v[k, pl.ds(_LANES * j, _LANES)] = (
          gidx_v[pl.ds(_CHUNK * k + _LANES * j, _LANES)])
    return carry
  lax.fori_loop(0, _NCHUNK, dst_fmt, 0)

  def zrow_init(i, carry):
    for j in range(_H // _LANES):
      zrow_v[i, pl.ds(_LANES * j, _LANES)] = zf
    return carry
  lax.fori_loop(0, 25, zrow_init, 0)

  if with_counts:
    def zcnt_init(i, carry):
      zcnt_v[i, :] = zf
      return carry
    lax.fori_loop(0, 125, zcnt_init, 0)

    ones = jnp.ones((_LANES,), jnp.float32)
    def ones_init(i, carry):
      ones_v[i, :] = ones
      return carry
    lax.fori_loop(0, _CHUNK, ones_init, 0)

  rbase = s * _RPT
  for ti in range(2):
    t = c + 2 * ti
    toff = t * _N

    def zacc(k, carry):
      pltpu.sync_copy(zrow_v, acc_s.at[pl.ds(rbase + 25 * k, 25)])
      return carry
    lax.fori_loop(0, _RPT // 25, zacc, 0)

    if with_counts and ti == 0:
      @pl.when(c == 0)
      def _():
        def zc(k, carry):
          pltpu.sync_copy(zcnt_v, cnt_s.at[pl.ds(rbase + 125 * k, 125)])
          return carry
        lax.fori_loop(0, _RPT // 125, zc, 0)

    plsc.subcore_barrier()

    def gidx_body(j, carry):
      v = src_v[pl.ds(_LANES * j, _LANES)]
      gidx_v[pl.ds(_LANES * j, _LANES)] = v + toff
      return carry
    lax.fori_loop(0, _EPT // _LANES, gidx_body, 0)

    def chunk_body(k, carry):
      base = _CHUNK * k
      pltpu.async_copy(
          x_hbm.at[gidx_v.at[pl.ds(base, _CHUNK)]], rows_v, sem).wait()
      pltpu.sync_copy(rows_v, acc_s.at[dst2_v.at[k]], add=True)
      if with_counts and ti == 0:
        @pl.when(c == 0)
        def _():
          pltpu.sync_copy(ones_v, cnt_s.at[dst2_v.at[k]], add=True)
      return carry
    lax.fori_loop(0, _NCHUNK, chunk_body, 0)

    plsc.subcore_barrier()

    pltpu.sync_copy(acc_s.at[pl.ds(rbase, _RPT)],
                    agg_hbm.at[pl.ds(toff + rbase, _RPT)])
    if with_counts and ti == 0:
      @pl.when(c == 0)
      def _():
        pltpu.sync_copy(cnt_s.at[pl.ds(rbase, _RPT)],
                        cnt_hbm.at[pl.ds(rbase, _RPT)])
    if ti == 0:
      plsc.subcore_barrier()


def _make_sc_agg(with_counts):
  outs = [jax.ShapeDtypeStruct((_T * _N, _H), jnp.float32)]
  if with_counts:
    outs.append(jax.ShapeDtypeStruct((_N, _LANES), jnp.float32))
  scratch = [
      pltpu.VMEM((_EPT,), jnp.int32),
      pltpu.VMEM((_EPT,), jnp.int32),
      pltpu.VMEM((_NCHUNK, _CHUNK), jnp.int32),
      pltpu.VMEM((_CHUNK, _H), jnp.float32),
      pltpu.VMEM((_CHUNK, _LANES), jnp.float32),
      pltpu.VMEM((25, _H), jnp.float32),
      pltpu.VMEM((125, _LANES), jnp.float32),
      pltpu.VMEM_SHARED((_N, _H), jnp.float32),
      pltpu.VMEM_SHARED((_N, _LANES), jnp.float32),
      pltpu.SemaphoreType.DMA,
  ]
  return pl.kernel(
      functools.partial(_sc_agg_body, with_counts),
      out_type=tuple(outs) if with_counts else outs[0],
      mesh=plsc.VectorSubcoreMesh(core_axis_name="c", subcore_axis_name="s"),
      scratch_types=scratch,
  )


_sc_agg_counts = _make_sc_agg(True)
_sc_agg = _make_sc_agg(False)


_BN = 1000


def _tc1_body(aggx_ref, x_ref, cnt_ref, wl_ref, wr_ref, b_ref, g_ref, be_ref,
              out_ref):
  cnt = jnp.maximum(cnt_ref[:, 0:1], 1.0)
  for t in range(_T):
    mean = aggx_ref[t] / cnt
    lin = (jnp.dot(mean, wl_ref[...], preferred_element_type=jnp.float32)
           + jnp.dot(x_ref[t], wr_ref[...], preferred_element_type=jnp.float32)
           + b_ref[...])
    mu = jnp.mean(lin, axis=-1, keepdims=True)
    var = jnp.mean((lin - mu) * (lin - mu), axis=-1, keepdims=True)
    h = (lin - mu) * lax.rsqrt(var + 1e-5) * g_ref[...] + be_ref[...]
    out_ref[t] = jnp.maximum(h, 0.0)


def _tc1(aggx, x_seq, cnt16, wlT, wrT, b, g, be):
  grid = (_N // _BN,)
  tnh = pl.BlockSpec((_T, _BN, _H), lambda i: (0, i, 0))
  w2d = pl.BlockSpec((_H, _H), lambda i: (0, 0))
  v1d = pl.BlockSpec((1, _H), lambda i: (0, 0))
  return pl.pallas_call(
      _tc1_body,
      grid=grid,
      in_specs=[tnh, tnh,
                pl.BlockSpec((_BN, _LANES), lambda i: (i, 0)),
                w2d, w2d, v1d, v1d, v1d],
      out_specs=tnh,
      out_shape=jax.ShapeDtypeStruct((_T, _N, _H), jnp.float32),
  )(aggx, x_seq, cnt16, wlT, wrT, b, g, be)


def _tc2_body(aggh_ref, h1_ref, cnt_ref, wl_ref, wr_ref, b2_ref, g2_ref,
              be2_ref, wih_ref, whh_ref, bih_ref, bhh_ref, wa_ref, ba_ref,
              wc1_ref, bc1_ref, wc2_ref, bc2_ref, out_ref):
  cnt = jnp.maximum(cnt_ref[:, 0:1], 1.0)
  hs2 = []
  for t in range(_T):
    mean = aggh_ref[t] / cnt
    res = h1_ref[t]
    lin = (jnp.dot(mean, wl_ref[...], preferred_element_type=jnp.float32)
           + jnp.dot(res, wr_ref[...], preferred_element_type=jnp.float32)
           + b2_ref[...])
    mu = jnp.mean(lin, axis=-1, keepdims=True)
    var = jnp.mean((lin - mu) * (lin - mu), axis=-1, keepdims=True)
    h = (lin - mu) * lax.rsqrt(var + 1e-5) * g2_ref[...] + be2_ref[...]
    hs2.append(jnp.maximum(h, 0.0) + res)

  h = jnp.zeros((_BN, _H), jnp.float32)
  outs = []
  for t in range(_T):
    xt = hs2[t]
    gi = (jnp.dot(xt, wih_ref[...], preferred_element_type=jnp.float32)
          + bih_ref[...])
    gh = (jnp.dot(h, whh_ref[...], preferred_element_type=jnp.float32)
          + bhh_ref[...])
    r = jax.nn.sigmoid(gi[:, 0:_H] + gh[:, 0:_H])
    zg = jax.nn.sigmoid(gi[:, _H:2 * _H] + gh[:, _H:2 * _H])
    n = jnp.tanh(gi[:, 2 * _H:3 * _H] + r * gh[:, 2 * _H:3 * _H])
    h = (1.0 - zg) * n + zg * h
    outs.append(h)

  wa = wa_ref[...]
  ba = ba_ref[0, 0]
  logit = [jnp.sum(outs[t] * wa, axis=-1, keepdims=True) + ba
           for t in range(_T)]
  m = logit[0]
  for t in range(1, _T):
    m = jnp.maximum(m, logit[t])
  e = [jnp.exp(logit[t] - m) for t in range(_T)]
  ssum = e[0]
  for t in range(1, _T):
    ssum = ssum + e[t]
  z = outs[0] * (e[0] / ssum)
  for t in range(1, _T):
    z = z + outs[t] * (e[t] / ssum)

  z1 = jnp.maximum(
      jnp.dot(z, wc1_ref[...], preferred_element_type=jnp.float32)
      + bc1_ref[...], 0.0)
  out_ref[...] = (jnp.dot(z1, wc2_ref[...], preferred_element_type=jnp.float32)
                  + bc2_ref[...])


def _tc2(aggh, h1, cnt16, wlT, wrT, b2, g2, be2, wihT, whhT, bih, bhh, wa, ba,
         wc1T, bc1, wc2T, bc2):
  grid = (_N // _BN,)
  tnh = pl.BlockSpec((_T, _BN, _H), lambda i: (0, i, 0))
  w2d = pl.BlockSpec((_H, _H), lambda i: (0, 0))
  v1d = pl.BlockSpec((1, _H), lambda i: (0, 0))
  w3h = pl.BlockSpec((_H, 3 * _H), lambda i: (0, 0))
  v3h = pl.BlockSpec((1, 3 * _H), lambda i: (0, 0))
  return pl.pallas_call(
      _tc2_body,
      grid=grid,
      in_specs=[tnh, tnh,
                pl.BlockSpec((_BN, _LANES), lambda i: (i, 0)),
                w2d, w2d, v1d, v1d, v1d,
                w3h, w3h, v3h, v3h,
                v1d, pl.BlockSpec((1, 1), lambda i: (0, 0)),
                w2d, v1d, w2d, v1d],
      out_specs=pl.BlockSpec((_BN, _H), lambda i: (i, 0)),
      out_shape=jax.ShapeDtypeStruct((_N, _H), jnp.float32),
  )(aggh, h1, cnt16, wlT, wrT, b2, g2, be2, wihT, whhT, bih, bhh, wa, ba,
    wc1T, bc1, wc2T, bc2)


def kernel(x_seq, edge_index, Wl1, Wr1, b1, g1, be1, Wl2, Wr2, b2, g2, be2,
           w_ih, w_hh, b_ih, b_hh, Wa, ba, Wc1, bc1, Wc2, bc2):
  src = edge_index[0]
  dst = edge_index[1]
  x_flat = x_seq.reshape(_T * _N, _H)

  aggx, cnt16 = _sc_agg_counts(x_flat, src, dst)
  h1 = _tc1(aggx.reshape(_T, _N, _H), x_seq, cnt16,
            Wl1.T, Wr1.T, b1[None], g1[None], be1[None])

  aggh = _sc_agg(h1.reshape(_T * _N, _H), src, dst)

  wc1T = jnp.zeros((_H, _H), jnp.float32).at[:, :_H // 2].set(Wc1.T)
  bc1p = jnp.zeros((1, _H), jnp.float32).at[0, :_H // 2].set(bc1)
  wc2T = jnp.zeros((_H, _H), jnp.float32).at[:_H // 2, :2].set(Wc2.T)
  bc2p = jnp.zeros((1, _H), jnp.float32).at[0, :2].set(bc2)

  out128 = _tc2(aggh.reshape(_T, _N, _H), h1, cnt16,
                Wl2.T, Wr2.T, b2[None], g2[None], be2[None],
                w_ih.T, w_hh.T, b_ih[None], b_hh[None],
                Wa, ba.reshape(1, 1), wc1T, bc1p, wc2T, bc2p)
  return out128[:, :2]


# speedup vs baseline: 2.7674x; 2.7674x over previous
import functools

import jax
import jax.numpy as jnp
from jax import lax
from jax.experimental import pallas as pl
from jax.experimental.pallas import tpu as pltpu
from jax.experimental.pallas import tpu_sc as plsc

_N = 10000
_T = 4
_H = 128
_E = 320000

_NS = 16
_EPT = _E // _NS
_CHUNK = 80
_NCHUNK = _EPT // _CHUNK
_LANES = 16
_RQ = 624
_RTAIL = _N - _RQ * _NS
_CEPT = _E // (2 * _NS)
_CNCHUNK = _CEPT // _CHUNK


def _sc_agg_body(with_counts, x_hbm, src_hbm, dst_hbm, *refs):
  if with_counts:
    agg_hbm, cnt_hbm = refs[0], refs[1]
    refs = refs[2:]
  else:
    agg_hbm = refs[0]
    cnt_hbm = None
    refs = refs[1:]
  (sidx_v, didx_v, rows_v, zrow_v, acc_s, sem) = refs

  c = lax.axis_index("c")
  s = lax.axis_index("s")
  ebase = s * _EPT

  zf = jnp.zeros((_LANES,), jnp.float32)

  def zrow_init(i, carry):
    for j in range(_H // _LANES):
      zrow_v[i, pl.ds(_LANES * j, _LANES)] = zf
    return carry
  lax.fori_loop(0, 24, zrow_init, 0)

  rbase = s * _RQ
  tail = _RQ * _NS

  def zero_acc():
    def zacc(k, carry):
      pltpu.sync_copy(zrow_v, acc_s.at[pl.ds(rbase + 24 * k, 24)])
      return carry
    lax.fori_loop(0, _RQ // 24, zacc, 0)

    @pl.when(s == _NS - 1)
    def _():
      pltpu.sync_copy(zrow_v.at[pl.ds(0, _RTAIL)],
                      acc_s.at[pl.ds(tail, _RTAIL)])

  if with_counts:
    one = jnp.ones((_LANES,), jnp.float32)
    def ones_init(i, carry):
      for j in range(_H // _LANES):
        rows_v[i, pl.ds(_LANES * j, _LANES)] = one
      return carry
    lax.fori_loop(0, _CHUNK, ones_init, 0)

    zero_acc()
    plsc.subcore_barrier()

    cbase = c * (_E // 2) + s * _CEPT
    def cnt_body(k, carry):
      pltpu.sync_copy(dst_hbm.at[pl.ds(cbase + _CHUNK * k, _CHUNK)], didx_v)
      pltpu.sync_copy(rows_v, acc_s.at[didx_v], add=True)
      return carry
    lax.fori_loop(0, _CNCHUNK, cnt_body, 0)

    plsc.subcore_barrier()

    coff = c * _N
    pltpu.sync_copy(acc_s.at[pl.ds(rbase, _RQ)],
                    cnt_hbm.at[pl.ds(coff + rbase, _RQ)])

    @pl.when(s == _NS - 1)
    def _():
      pltpu.sync_copy(acc_s.at[pl.ds(tail, _RTAIL)],
                      cnt_hbm.at[pl.ds(coff + tail, _RTAIL)])

    plsc.subcore_barrier()

  for ti in range(2):
    t = c + 2 * ti
    toff = t * _N

    zero_acc()
    plsc.subcore_barrier()

    def chunk_body(k, carry):
      base = ebase + _CHUNK * k
      pltpu.sync_copy(src_hbm.at[pl.ds(base, _CHUNK)], sidx_v)
      pltpu.sync_copy(dst_hbm.at[pl.ds(base, _CHUNK)], didx_v)
      for j in range(_CHUNK // _LANES):
        sidx_v[pl.ds(_LANES * j, _LANES)] = (
            sidx_v[pl.ds(_LANES * j, _LANES)] + toff)
      pltpu.async_copy(x_hbm.at[sidx_v], rows_v, sem).wait()
      pltpu.sync_copy(rows_v, acc_s.at[didx_v], add=True)
      return carry
    lax.fori_loop(0, _NCHUNK, chunk_body, 0)

    plsc.subcore_barrier()

    pltpu.sync_copy(acc_s.at[pl.ds(rbase, _RQ)],
                    agg_hbm.at[pl.ds(toff + rbase, _RQ)])

    @pl.when(s == _NS - 1)
    def _():
      pltpu.sync_copy(acc_s.at[pl.ds(tail, _RTAIL)],
                      agg_hbm.at[pl.ds(toff + tail, _RTAIL)])

    if ti == 0:
      plsc.subcore_barrier()


@functools.lru_cache(maxsize=None)
def _make_sc_agg(with_counts):
  outs = [jax.ShapeDtypeStruct((_T * _N, _H), jnp.float32)]
  if with_counts:
    outs.append(jax.ShapeDtypeStruct((2 * _N, _H), jnp.float32))
  scratch = [
      pltpu.VMEM((_CHUNK,), jnp.int32),
      pltpu.VMEM((_CHUNK,), jnp.int32),
      pltpu.VMEM((_CHUNK, _H), jnp.float32),
      pltpu.VMEM((24, _H), jnp.float32),
      pltpu.VMEM_SHARED((_N, _H), jnp.float32),
      pltpu.SemaphoreType.DMA,
  ]
  return pl.kernel(
      functools.partial(_sc_agg_body, with_counts),
      out_type=tuple(outs) if with_counts else outs[0],
      mesh=plsc.VectorSubcoreMesh(core_axis_name="c", subcore_axis_name="s",
                                  num_cores=2, num_subcores=_NS),
      scratch_types=scratch,
  )


_BN = 1000


def _tc1_body(aggx_ref, x_ref, cnt_ref, wl_ref, wr_ref, b_ref, g_ref, be_ref,
              out_ref):
  cnt = jnp.maximum(cnt_ref[0, :, 0:1] + cnt_ref[1, :, 0:1], 1.0)
  for t in range(_T):
    mean = aggx_ref[t] / cnt
    lin = (jnp.dot(mean, wl_ref[...], preferred_element_type=jnp.float32, precision=lax.Precision.HIGHEST)
           + jnp.dot(x_ref[t], wr_ref[...], preferred_element_type=jnp.float32, precision=lax.Precision.HIGHEST)
           + b_ref[...])
    mu = jnp.mean(lin, axis=-1, keepdims=True)
    var = jnp.mean((lin - mu) * (lin - mu), axis=-1, keepdims=True)
    h = (lin - mu) * lax.rsqrt(var + 1e-5) * g_ref[...] + be_ref[...]
    out_ref[t] = jnp.maximum(h, 0.0)


def _tc1(aggx, x_seq, cnt2, wlT, wrT, b, g, be):
  grid = (_N // _BN,)
  tnh = pl.BlockSpec((_T, _BN, _H), lambda i: (0, i, 0))
  cnh = pl.BlockSpec((2, _BN, _H), lambda i: (0, i, 0))
  w2d = pl.BlockSpec((_H, _H), lambda i: (0, 0))
  v1d = pl.BlockSpec((1, _H), lambda i: (0, 0))
  return pl.pallas_call(
      _tc1_body,
      grid=grid,
      in_specs=[tnh, tnh, cnh, w2d, w2d, v1d, v1d, v1d],
      out_specs=tnh,
      out_shape=jax.ShapeDtypeStruct((_T, _N, _H), jnp.float32),
  )(aggx, x_seq, cnt2, wlT, wrT, b, g, be)


def _tc2_body(aggh_ref, h1_ref, cnt_ref, wl_ref, wr_ref, b2_ref, g2_ref,
              be2_ref, wih_ref, whh_ref, bih_ref, bhh_ref, wa_ref, ba_ref,
              wc1_ref, bc1_ref, wc2_ref, bc2_ref, out_ref):
  cnt = jnp.maximum(cnt_ref[0, :, 0:1] + cnt_ref[1, :, 0:1], 1.0)
  hs2 = []
  for t in range(_T):
    mean = aggh_ref[t] / cnt
    res = h1_ref[t]
    lin = (jnp.dot(mean, wl_ref[...], preferred_element_type=jnp.float32, precision=lax.Precision.HIGHEST)
           + jnp.dot(res, wr_ref[...], preferred_element_type=jnp.float32, precision=lax.Precision.HIGHEST)
           + b2_ref[...])
    mu = jnp.mean(lin, axis=-1, keepdims=True)
    var = jnp.mean((lin - mu) * (lin - mu), axis=-1, keepdims=True)
    h = (lin - mu) * lax.rsqrt(var + 1e-5) * g2_ref[...] + be2_ref[...]
    hs2.append(jnp.maximum(h, 0.0) + res)

  h = jnp.zeros((_BN, _H), jnp.float32)
  outs = []
  for t in range(_T):
    xt = hs2[t]
    gi = (jnp.dot(xt, wih_ref[...], preferred_element_type=jnp.float32, precision=lax.Precision.HIGHEST)
          + bih_ref[...])
    gh = (jnp.dot(h, whh_ref[...], preferred_element_type=jnp.float32, precision=lax.Precision.HIGHEST)
          + bhh_ref[...])
    r = jax.nn.sigmoid(gi[:, 0:_H] + gh[:, 0:_H])
    zg = jax.nn.sigmoid(gi[:, _H:2 * _H] + gh[:, _H:2 * _H])
    n = jnp.tanh(gi[:, 2 * _H:3 * _H] + r * gh[:, 2 * _H:3 * _H])
    h = (1.0 - zg) * n + zg * h
    outs.append(h)

  wa = wa_ref[...]
  ba = ba_ref[0, 0]
  logit = [jnp.sum(outs[t] * wa, axis=-1, keepdims=True) + ba
           for t in range(_T)]
  m = logit[0]
  for t in range(1, _T):
    m = jnp.maximum(m, logit[t])
  e = [jnp.exp(logit[t] - m) for t in range(_T)]
  ssum = e[0]
  for t in range(1, _T):
    ssum = ssum + e[t]
  z = outs[0] * (e[0] / ssum)
  for t in range(1, _T):
    z = z + outs[t] * (e[t] / ssum)

  z1 = jnp.maximum(
      jnp.dot(z, wc1_ref[...], preferred_element_type=jnp.float32, precision=lax.Precision.HIGHEST)
      + bc1_ref[...], 0.0)
  out_ref[...] = (jnp.dot(z1, wc2_ref[...], preferred_element_type=jnp.float32, precision=lax.Precision.HIGHEST)
                  + bc2_ref[...])


def _tc2(aggh, h1, cnt2, wlT, wrT, b2, g2, be2, wihT, whhT, bih, bhh, wa, ba,
         wc1T, bc1, wc2T, bc2):
  grid = (_N // _BN,)
  tnh = pl.BlockSpec((_T, _BN, _H), lambda i: (0, i, 0))
  cnh = pl.BlockSpec((2, _BN, _H), lambda i: (0, i, 0))
  w2d = pl.BlockSpec((_H, _H), lambda i: (0, 0))
  v1d = pl.BlockSpec((1, _H), lambda i: (0, 0))
  w3h = pl.BlockSpec((_H, 3 * _H), lambda i: (0, 0))
  v3h = pl.BlockSpec((1, 3 * _H), lambda i: (0, 0))
  return pl.pallas_call(
      _tc2_body,
      grid=grid,
      in_specs=[tnh, tnh, cnh,
                w2d, w2d, v1d, v1d, v1d,
                w3h, w3h, v3h, v3h,
                v1d, pl.BlockSpec((1, 1), lambda i: (0, 0)),
                w2d, v1d, w2d, v1d],
      out_specs=pl.BlockSpec((_BN, _H), lambda i: (i, 0)),
      out_shape=jax.ShapeDtypeStruct((_N, _H), jnp.float32),
  )(aggh, h1, cnt2, wlT, wrT, b2, g2, be2, wihT, whhT, bih, bhh, wa, ba,
    wc1T, bc1, wc2T, bc2)


def kernel(x_seq, edge_index, Wl1, Wr1, b1, g1, be1, Wl2, Wr2, b2, g2, be2,
           w_ih, w_hh, b_ih, b_hh, Wa, ba, Wc1, bc1, Wc2, bc2):
  src = edge_index[0]
  dst = edge_index[1]
  x_flat = x_seq.reshape(_T * _N, _H)

  aggx, cnt2 = _make_sc_agg(True)(x_flat, src, dst)
  cnt2 = cnt2.reshape(2, _N, _H)
  h1 = _tc1(aggx.reshape(_T, _N, _H), x_seq, cnt2,
            Wl1.T, Wr1.T, b1[None], g1[None], be1[None])

  aggh = _make_sc_agg(False)(h1.reshape(_T * _N, _H), src, dst)

  wc1T = jnp.zeros((_H, _H), jnp.float32).at[:, :_H // 2].set(Wc1.T)
  bc1p = jnp.zeros((1, _H), jnp.float32).at[0, :_H // 2].set(bc1)
  wc2T = jnp.zeros((_H, _H), jnp.float32).at[:_H // 2, :2].set(Wc2.T)
  bc2p = jnp.zeros((1, _H), jnp.float32).at[0, :2].set(bc2)

  out128 = _tc2(aggh.reshape(_T, _N, _H), h1, cnt2,
                Wl2.T, Wr2.T, b2[None], g2[None], be2[None],
                w_ih.T, w_hh.T, b_ih[None], b_hh[None],
                Wa, ba.reshape(1, 1), wc1T, bc1p, wc2T, bc2p)
  return out128[:, :2]


# speedup vs baseline: 5.0304x; 1.8178x over previous
import functools

import jax
import jax.numpy as jnp
from jax import lax
from jax.experimental import pallas as pl
from jax.experimental.pallas import tpu as pltpu
from jax.experimental.pallas import tpu_sc as plsc

_N = 10000
_T = 4
_H = 128
_E = 320000

_NS = 16
_EPT = _E // _NS
_CHUNK = 80
_NCHUNK = _EPT // _CHUNK
_LANES = 16
_RQ = 624
_RTAIL = _N - _RQ * _NS
_CEPT = _E // (2 * _NS)
_CNCHUNK = _CEPT // _CHUNK


def _sc_agg_body(with_counts, x_hbm, src_hbm, dst_hbm, *refs):
  if with_counts:
    agg_hbm, cnt_hbm = refs[0], refs[1]
    refs = refs[2:]
  else:
    agg_hbm = refs[0]
    cnt_hbm = None
    refs = refs[1:]
  (sidx0, sidx1, didx0, didx1, rows0, rows1, zrow_v, acc_s,
   issem0, issem1, idsem0, idsem1, gsem0, gsem1) = refs
  sidx = (sidx0, sidx1)
  didx = (didx0, didx1)
  rows = (rows0, rows1)
  issem = (issem0, issem1)
  idsem = (idsem0, idsem1)
  gsem = (gsem0, gsem1)

  c = lax.axis_index("c")
  s = lax.axis_index("s")
  ebase = s * _EPT

  zf = jnp.zeros((_LANES,), jnp.float32)

  def zrow_init(i, carry):
    for j in range(_H // _LANES):
      zrow_v[i, pl.ds(_LANES * j, _LANES)] = zf
    return carry
  lax.fori_loop(0, 24, zrow_init, 0)

  rbase = s * _RQ
  tail = _RQ * _NS

  def zero_acc():
    def zacc(k, carry):
      pltpu.sync_copy(zrow_v, acc_s.at[pl.ds(rbase + 24 * k, 24)])
      return carry
    lax.fori_loop(0, _RQ // 24, zacc, 0)

    @pl.when(s == _NS - 1)
    def _():
      pltpu.sync_copy(zrow_v.at[pl.ds(0, _RTAIL)],
                      acc_s.at[pl.ds(tail, _RTAIL)])

  if with_counts:
    one = jnp.ones((_LANES,), jnp.float32)
    def ones_init(i, carry):
      for j in range(_H // _LANES):
        rows0[i, pl.ds(_LANES * j, _LANES)] = one
      return carry
    lax.fori_loop(0, _CHUNK, ones_init, 0)

    zero_acc()
    plsc.subcore_barrier()

    cbase = c * (_E // 2) + s * _CEPT

    def cidx_start(k, b):
      o = cbase + _CHUNK * lax.rem(k, _CNCHUNK)
      pltpu.async_copy(dst_hbm.at[pl.ds(o, _CHUNK)], didx[b], idsem[b])

    def cidx_wait(b):
      pltpu.make_async_copy(dst_hbm.at[pl.ds(0, _CHUNK)], didx[b],
                            idsem[b]).wait()

    cidx_start(0, 0)
    cidx_start(1, 1)

    def cnt_body(m, carry):
      for b in range(2):
        k = 2 * m + b
        cidx_wait(b)
        pltpu.sync_copy(rows0, acc_s.at[didx[b]], add=True)
        cidx_start(k + 2, b)
      return carry
    lax.fori_loop(0, (_CNCHUNK - 1) // 2, cnt_body, 0)

    cidx_wait(0)
    pltpu.sync_copy(rows0, acc_s.at[didx0], add=True)
    cidx_wait(1)

    plsc.subcore_barrier()

    coff = c * _N
    pltpu.sync_copy(acc_s.at[pl.ds(rbase, _RQ)],
                    cnt_hbm.at[pl.ds(coff + rbase, _RQ)])

    @pl.when(s == _NS - 1)
    def _():
      pltpu.sync_copy(acc_s.at[pl.ds(tail, _RTAIL)],
                      cnt_hbm.at[pl.ds(coff + tail, _RTAIL)])

    plsc.subcore_barrier()

  for ti in range(2):
    t = c + 2 * ti
    toff = t * _N

    zero_acc()
    plsc.subcore_barrier()

    def idx_start(k, b):
      o = ebase + _CHUNK * lax.rem(k, _NCHUNK)
      pltpu.async_copy(src_hbm.at[pl.ds(o, _CHUNK)], sidx[b], issem[b])
      pltpu.async_copy(dst_hbm.at[pl.ds(o, _CHUNK)], didx[b], idsem[b])

    def idx_wait(b):
      pltpu.make_async_copy(src_hbm.at[pl.ds(0, _CHUNK)], sidx[b],
                            issem[b]).wait()
      pltpu.make_async_copy(dst_hbm.at[pl.ds(0, _CHUNK)], didx[b],
                            idsem[b]).wait()

    def add_toff(b):
      for j in range(_CHUNK // _LANES):
        sidx[b][pl.ds(_LANES * j, _LANES)] = (
            sidx[b][pl.ds(_LANES * j, _LANES)] + toff)

    def gather_start(b):
      pltpu.async_copy(x_hbm.at[sidx[b]], rows[b], gsem[b])

    def gather_wait(b):
      pltpu.make_async_copy(x_hbm.at[sidx[b]], rows[b], gsem[b]).wait()

    idx_start(0, 0)
    idx_start(1, 1)
    idx_wait(0)
    add_toff(0)
    gather_start(0)

    def chunk_body(m, carry):
      for b in range(2):
        k = 2 * m + b
        nb = 1 - b
        idx_wait(nb)
        add_toff(nb)
        gather_wait(b)
        gather_start(nb)
        pltpu.sync_copy(rows[b], acc_s.at[didx[b]], add=True)
        idx_start(k + 2, b)
      return carry
    lax.fori_loop(0, _NCHUNK // 2, chunk_body, 0)

    gather_wait(0)
    idx_wait(1)

    plsc.subcore_barrier()

    pltpu.sync_copy(acc_s.at[pl.ds(rbase, _RQ)],
                    agg_hbm.at[pl.ds(toff + rbase, _RQ)])

    @pl.when(s == _NS - 1)
    def _():
      pltpu.sync_copy(acc_s.at[pl.ds(tail, _RTAIL)],
                      agg_hbm.at[pl.ds(toff + tail, _RTAIL)])

    if ti == 0:
      plsc.subcore_barrier()


@functools.lru_cache(maxsize=None)
def _make_sc_agg(with_counts):
  outs = [jax.ShapeDtypeStruct((_T * _N, _H), jnp.float32)]
  if with_counts:
    outs.append(jax.ShapeDtypeStruct((2 * _N, _H), jnp.float32))
  scratch = [
      pltpu.VMEM((_CHUNK,), jnp.int32),
      pltpu.VMEM((_CHUNK,), jnp.int32),
      pltpu.VMEM((_CHUNK,), jnp.int32),
      pltpu.VMEM((_CHUNK,), jnp.int32),
      pltpu.VMEM((_CHUNK, _H), jnp.float32),
      pltpu.VMEM((_CHUNK, _H), jnp.float32),
      pltpu.VMEM((24, _H), jnp.float32),
      pltpu.VMEM_SHARED((_N, _H), jnp.float32),
      pltpu.SemaphoreType.DMA,
      pltpu.SemaphoreType.DMA,
      pltpu.SemaphoreType.DMA,
      pltpu.SemaphoreType.DMA,
      pltpu.SemaphoreType.DMA,
      pltpu.SemaphoreType.DMA,
  ]
  return pl.kernel(
      functools.partial(_sc_agg_body, with_counts),
      out_type=tuple(outs) if with_counts else outs[0],
      mesh=plsc.VectorSubcoreMesh(core_axis_name="c", subcore_axis_name="s",
                                  num_cores=2, num_subcores=_NS),
      scratch_types=scratch,
  )


_BN = 1000


def _tc1_body(aggx_ref, x_ref, cnt_ref, wl_ref, wr_ref, b_ref, g_ref, be_ref,
              out_ref):
  cnt = jnp.maximum(cnt_ref[0, :, 0:1] + cnt_ref[1, :, 0:1], 1.0)
  for t in range(_T):
    mean = aggx_ref[t] / cnt
    lin = (jnp.dot(mean, wl_ref[...], preferred_element_type=jnp.float32, precision=lax.Precision.HIGHEST)
           + jnp.dot(x_ref[t], wr_ref[...], preferred_element_type=jnp.float32, precision=lax.Precision.HIGHEST)
           + b_ref[...])
    mu = jnp.mean(lin, axis=-1, keepdims=True)
    var = jnp.mean((lin - mu) * (lin - mu), axis=-1, keepdims=True)
    h = (lin - mu) * lax.rsqrt(var + 1e-5) * g_ref[...] + be_ref[...]
    out_ref[t] = jnp.maximum(h, 0.0)


def _tc1(aggx, x_seq, cnt2, wlT, wrT, b, g, be):
  grid = (_N // _BN,)
  tnh = pl.BlockSpec((_T, _BN, _H), lambda i: (0, i, 0))
  cnh = pl.BlockSpec((2, _BN, _H), lambda i: (0, i, 0))
  w2d = pl.BlockSpec((_H, _H), lambda i: (0, 0))
  v1d = pl.BlockSpec((1, _H), lambda i: (0, 0))
  return pl.pallas_call(
      _tc1_body,
      grid=grid,
      in_specs=[tnh, tnh, cnh, w2d, w2d, v1d, v1d, v1d],
      out_specs=tnh,
      out_shape=jax.ShapeDtypeStruct((_T, _N, _H), jnp.float32),
  )(aggx, x_seq, cnt2, wlT, wrT, b, g, be)


def _tc2_body(aggh_ref, h1_ref, cnt_ref, wl_ref, wr_ref, b2_ref, g2_ref,
              be2_ref, wih_ref, whh_ref, bih_ref, bhh_ref, wa_ref, ba_ref,
              wc1_ref, bc1_ref, wc2_ref, bc2_ref, out_ref):
  cnt = jnp.maximum(cnt_ref[0, :, 0:1] + cnt_ref[1, :, 0:1], 1.0)
  hs2 = []
  for t in range(_T):
    mean = aggh_ref[t] / cnt
    res = h1_ref[t]
    lin = (jnp.dot(mean, wl_ref[...], preferred_element_type=jnp.float32, precision=lax.Precision.HIGHEST)
           + jnp.dot(res, wr_ref[...], preferred_element_type=jnp.float32, precision=lax.Precision.HIGHEST)
           + b2_ref[...])
    mu = jnp.mean(lin, axis=-1, keepdims=True)
    var = jnp.mean((lin - mu) * (lin - mu), axis=-1, keepdims=True)
    h = (lin - mu) * lax.rsqrt(var + 1e-5) * g2_ref[...] + be2_ref[...]
    hs2.append(jnp.maximum(h, 0.0) + res)

  h = jnp.zeros((_BN, _H), jnp.float32)
  outs = []
  for t in range(_T):
    xt = hs2[t]
    gi = (jnp.dot(xt, wih_ref[...], preferred_element_type=jnp.float32, precision=lax.Precision.HIGHEST)
          + bih_ref[...])
    gh = (jnp.dot(h, whh_ref[...], preferred_element_type=jnp.float32, precision=lax.Precision.HIGHEST)
          + bhh_ref[...])
    r = jax.nn.sigmoid(gi[:, 0:_H] + gh[:, 0:_H])
    zg = jax.nn.sigmoid(gi[:, _H:2 * _H] + gh[:, _H:2 * _H])
    n = jnp.tanh(gi[:, 2 * _H:3 * _H] + r * gh[:, 2 * _H:3 * _H])
    h = (1.0 - zg) * n + zg * h
    outs.append(h)

  wa = wa_ref[...]
  ba = ba_ref[0, 0]
  logit = [jnp.sum(outs[t] * wa, axis=-1, keepdims=True) + ba
           for t in range(_T)]
  m = logit[0]
  for t in range(1, _T):
    m = jnp.maximum(m, logit[t])
  e = [jnp.exp(logit[t] - m) for t in range(_T)]
  ssum = e[0]
  for t in range(1, _T):
    ssum = ssum + e[t]
  z = outs[0] * (e[0] / ssum)
  for t in range(1, _T):
    z = z + outs[t] * (e[t] / ssum)

  z1 = jnp.maximum(
      jnp.dot(z, wc1_ref[...], preferred_element_type=jnp.float32, precision=lax.Precision.HIGHEST)
      + bc1_ref[...], 0.0)
  out_ref[...] = (jnp.dot(z1, wc2_ref[...], preferred_element_type=jnp.float32, precision=lax.Precision.HIGHEST)
                  + bc2_ref[...])


def _tc2(aggh, h1, cnt2, wlT, wrT, b2, g2, be2, wihT, whhT, bih, bhh, wa, ba,
         wc1T, bc1, wc2T, bc2):
  grid = (_N // _BN,)
  tnh = pl.BlockSpec((_T, _BN, _H), lambda i: (0, i, 0))
  cnh = pl.BlockSpec((2, _BN, _H), lambda i: (0, i, 0))
  w2d = pl.BlockSpec((_H, _H), lambda i: (0, 0))
  v1d = pl.BlockSpec((1, _H), lambda i: (0, 0))
  w3h = pl.BlockSpec((_H, 3 * _H), lambda i: (0, 0))
  v3h = pl.BlockSpec((1, 3 * _H), lambda i: (0, 0))
  return pl.pallas_call(
      _tc2_body,
      grid=grid,
      in_specs=[tnh, tnh, cnh,
                w2d, w2d, v1d, v1d, v1d,
                w3h, w3h, v3h, v3h,
                v1d, pl.BlockSpec((1, 1), lambda i: (0, 0)),
                w2d, v1d, w2d, v1d],
      out_specs=pl.BlockSpec((_BN, _H), lambda i: (i, 0)),
      out_shape=jax.ShapeDtypeStruct((_N, _H), jnp.float32),
  )(aggh, h1, cnt2, wlT, wrT, b2, g2, be2, wihT, whhT, bih, bhh, wa, ba,
    wc1T, bc1, wc2T, bc2)


def kernel(x_seq, edge_index, Wl1, Wr1, b1, g1, be1, Wl2, Wr2, b2, g2, be2,
           w_ih, w_hh, b_ih, b_hh, Wa, ba, Wc1, bc1, Wc2, bc2):
  src = edge_index[0]
  dst = edge_index[1]
  x_flat = x_seq.reshape(_T * _N, _H)

  aggx, cnt2 = _make_sc_agg(True)(x_flat, src, dst)
  cnt2 = cnt2.reshape(2, _N, _H)
  h1 = _tc1(aggx.reshape(_T, _N, _H), x_seq, cnt2,
            Wl1.T, Wr1.T, b1[None], g1[None], be1[None])

  aggh = _make_sc_agg(False)(h1.reshape(_T * _N, _H), src, dst)

  wc1T = jnp.zeros((_H, _H), jnp.float32).at[:, :_H // 2].set(Wc1.T)
  bc1p = jnp.zeros((1, _H), jnp.float32).at[0, :_H // 2].set(bc1)
  wc2T = jnp.zeros((_H, _H), jnp.float32).at[:_H // 2, :2].set(Wc2.T)
  bc2p = jnp.zeros((1, _H), jnp.float32).at[0, :2].set(bc2)

  out128 = _tc2(aggh.reshape(_T, _N, _H), h1, cnt2,
                Wl2.T, Wr2.T, b2[None], g2[None], be2[None],
                w_ih.T, w_hh.T, b_ih[None], b_hh[None],
                Wa, ba.reshape(1, 1), wc1T, bc1p, wc2T, bc2p)
  return out128[:, :2]


# speedup vs baseline: 5.0407x; 1.0020x over previous
import functools

import jax
import jax.numpy as jnp
from jax import lax
from jax.experimental import pallas as pl
from jax.experimental.pallas import tpu as pltpu
from jax.experimental.pallas import tpu_sc as plsc

_N = 10000
_T = 4
_H = 128
_E = 320000

_NS = 16
_EPT = _E // _NS
_CHUNK = 80
_NCHUNK = _EPT // _CHUNK
_LANES = 16
_RQ = 624
_RTAIL = _N - _RQ * _NS
_CEPT = _E // (2 * _NS)
_CNCHUNK = _CEPT // _CHUNK


def _sc_agg_body(with_counts, x_hbm, src_hbm, dst_hbm, *refs):
  if with_counts:
    agg_hbm, cnt_hbm = refs[0], refs[1]
    refs = refs[2:]
  else:
    agg_hbm = refs[0]
    cnt_hbm = None
    refs = refs[1:]
  (sidx0, sidx1, didx0, didx1, sdidx0, sdidx1, rows0, rows1, zrow_v, acc_s,
   issem0, issem1, idsem0, idsem1, gsem0, gsem1, ssem0, ssem1) = refs
  sidx = (sidx0, sidx1)
  didx = (didx0, didx1)
  sdidx = (sdidx0, sdidx1)
  rows = (rows0, rows1)
  issem = (issem0, issem1)
  idsem = (idsem0, idsem1)
  gsem = (gsem0, gsem1)
  ssem = (ssem0, ssem1)

  c = lax.axis_index("c")
  s = lax.axis_index("s")
  ebase = s * _EPT

  zf = jnp.zeros((_LANES,), jnp.float32)

  def zrow_init(i, carry):
    for j in range(_H // _LANES):
      zrow_v[i, pl.ds(_LANES * j, _LANES)] = zf
    return carry
  lax.fori_loop(0, 24, zrow_init, 0)

  rbase = s * _RQ
  tail = _RQ * _NS

  def zero_acc():
    def zacc(k, carry):
      pltpu.sync_copy(zrow_v, acc_s.at[pl.ds(rbase + 24 * k, 24)])
      return carry
    lax.fori_loop(0, _RQ // 24, zacc, 0)

    @pl.when(s == _NS - 1)
    def _():
      pltpu.sync_copy(zrow_v.at[pl.ds(0, _RTAIL)],
                      acc_s.at[pl.ds(tail, _RTAIL)])

  if with_counts:
    one = jnp.ones((_LANES,), jnp.float32)
    def ones_init(i, carry):
      for j in range(_H // _LANES):
        rows0[i, pl.ds(_LANES * j, _LANES)] = one
      return carry
    lax.fori_loop(0, _CHUNK, ones_init, 0)

    zero_acc()
    plsc.subcore_barrier()

    cbase = c * (_E // 2) + s * _CEPT

    def cidx_start(k, b):
      o = cbase + _CHUNK * lax.rem(k, _CNCHUNK)
      pltpu.async_copy(dst_hbm.at[pl.ds(o, _CHUNK)], didx[b], idsem[b])

    def cidx_wait(b):
      pltpu.make_async_copy(dst_hbm.at[pl.ds(0, _CHUNK)], didx[b],
                            idsem[b]).wait()

    cidx_start(0, 0)
    cidx_start(1, 1)

    def cnt_body(m, carry):
      for b in range(2):
        k = 2 * m + b
        cidx_wait(b)
        pltpu.sync_copy(rows0, acc_s.at[didx[b]], add=True)
        cidx_start(k + 2, b)
      return carry
    lax.fori_loop(0, (_CNCHUNK - 1) // 2, cnt_body, 0)

    cidx_wait(0)
    pltpu.sync_copy(rows0, acc_s.at[didx0], add=True)
    cidx_wait(1)

    plsc.subcore_barrier()

    coff = c * _N
    pltpu.sync_copy(acc_s.at[pl.ds(rbase, _RQ)],
                    cnt_hbm.at[pl.ds(coff + rbase, _RQ)])

    @pl.when(s == _NS - 1)
    def _():
      pltpu.sync_copy(acc_s.at[pl.ds(tail, _RTAIL)],
                      cnt_hbm.at[pl.ds(coff + tail, _RTAIL)])

    plsc.subcore_barrier()

  for ti in range(2):
    t = c + 2 * ti
    toff = t * _N

    zero_acc()
    plsc.subcore_barrier()

    def idx_start(k, b):
      o = ebase + _CHUNK * lax.rem(k, _NCHUNK)
      pltpu.async_copy(src_hbm.at[pl.ds(o, _CHUNK)], sidx[b], issem[b])
      pltpu.async_copy(dst_hbm.at[pl.ds(o, _CHUNK)], didx[b], idsem[b])

    def idx_wait(b):
      pltpu.make_async_copy(src_hbm.at[pl.ds(0, _CHUNK)], sidx[b],
                            issem[b]).wait()
      pltpu.make_async_copy(dst_hbm.at[pl.ds(0, _CHUNK)], didx[b],
                            idsem[b]).wait()

    def add_toff(b):
      for j in range(_CHUNK // _LANES):
        sidx[b][pl.ds(_LANES * j, _LANES)] = (
            sidx[b][pl.ds(_LANES * j, _LANES)] + toff)

    def gather_start(b):
      pltpu.async_copy(x_hbm.at[sidx[b]], rows[b], gsem[b])

    def gather_wait(b):
      pltpu.make_async_copy(x_hbm.at[sidx[b]], rows[b], gsem[b]).wait()

    def copy_didx(b):
      for j in range(_CHUNK // _LANES):
        sdidx[b][pl.ds(_LANES * j, _LANES)] = didx[b][pl.ds(_LANES * j, _LANES)]

    def scatter_start(b):
      pltpu.async_copy(rows[b], acc_s.at[sdidx[b]], ssem[b], add=True)

    def scatter_wait(b):
      pltpu.make_async_copy(rows[b], acc_s.at[sdidx[b]], ssem[b]).wait()

    idx_start(0, 0)
    idx_start(1, 1)
    idx_wait(0)
    add_toff(0)
    gather_start(0)
    idx_wait(1)
    add_toff(1)
    gather_wait(0)
    gather_start(1)
    copy_didx(0)
    scatter_start(0)
    idx_start(2, 0)

    def chunk_body(m, carry):
      for b in (1, 0):
        k = 2 * m + 2 - b
        nb = 1 - b
        idx_wait(nb)
        add_toff(nb)
        gather_wait(b)
        scatter_wait(nb)
        gather_start(nb)
        copy_didx(b)
        scatter_start(b)
        idx_start(k + 2, b)
      return carry
    lax.fori_loop(0, _NCHUNK // 2 - 1, chunk_body, 0)

    idx_wait(0)
    gather_wait(1)
    scatter_wait(0)
    copy_didx(1)
    scatter_start(1)
    scatter_wait(1)

    plsc.subcore_barrier()

    pltpu.sync_copy(acc_s.at[pl.ds(rbase, _RQ)],
                    agg_hbm.at[pl.ds(toff + rbase, _RQ)])

    @pl.when(s == _NS - 1)
    def _():
      pltpu.sync_copy(acc_s.at[pl.ds(tail, _RTAIL)],
                      agg_hbm.at[pl.ds(toff + tail, _RTAIL)])

    if ti == 0:
      plsc.subcore_barrier()


@functools.lru_cache(maxsize=None)
def _make_sc_agg(with_counts):
  outs = [jax.ShapeDtypeStruct((_T * _N, _H), jnp.float32)]
  if with_counts:
    outs.append(jax.ShapeDtypeStruct((2 * _N, _H), jnp.float32))
  scratch = [
      pltpu.VMEM((_CHUNK,), jnp.int32),
      pltpu.VMEM((_CHUNK,), jnp.int32),
      pltpu.VMEM((_CHUNK,), jnp.int32),
      pltpu.VMEM((_CHUNK,), jnp.int32),
      pltpu.VMEM((_CHUNK,), jnp.int32),
      pltpu.VMEM((_CHUNK,), jnp.int32),
      pltpu.VMEM((_CHUNK, _H), jnp.float32),
      pltpu.VMEM((_CHUNK, _H), jnp.float32),
      pltpu.VMEM((24, _H), jnp.float32),
      pltpu.VMEM_SHARED((_N, _H), jnp.float32),
      pltpu.SemaphoreType.DMA,
      pltpu.SemaphoreType.DMA,
      pltpu.SemaphoreType.DMA,
      pltpu.SemaphoreType.DMA,
      pltpu.SemaphoreType.DMA,
      pltpu.SemaphoreType.DMA,
      pltpu.SemaphoreType.DMA,
      pltpu.SemaphoreType.DMA,
  ]
  return pl.kernel(
      functools.partial(_sc_agg_body, with_counts),
      out_type=tuple(outs) if with_counts else outs[0],
      mesh=plsc.VectorSubcoreMesh(core_axis_name="c", subcore_axis_name="s",
                                  num_cores=2, num_subcores=_NS),
      scratch_types=scratch,
  )


_BN = 1000


def _tc1_body(aggx_ref, x_ref, cnt_ref, wl_ref, wr_ref, b_ref, g_ref, be_ref,
              out_ref):
  cnt = jnp.maximum(cnt_ref[0, :, 0:1] + cnt_ref[1, :, 0:1], 1.0)
  for t in range(_T):
    mean = aggx_ref[t] / cnt
    lin = (jnp.dot(mean, wl_ref[...], preferred_element_type=jnp.float32, precision=lax.Precision.HIGHEST)
           + jnp.dot(x_ref[t], wr_ref[...], preferred_element_type=jnp.float32, precision=lax.Precision.HIGHEST)
           + b_ref[...])
    mu = jnp.mean(lin, axis=-1, keepdims=True)
    var = jnp.mean((lin - mu) * (lin - mu), axis=-1, keepdims=True)
    h = (lin - mu) * lax.rsqrt(var + 1e-5) * g_ref[...] + be_ref[...]
    out_ref[t] = jnp.maximum(h, 0.0)


def _tc1(aggx, x_seq, cnt2, wlT, wrT, b, g, be):
  grid = (_N // _BN,)
  tnh = pl.BlockSpec((_T, _BN, _H), lambda i: (0, i, 0))
  cnh = pl.BlockSpec((2, _BN, _H), lambda i: (0, i, 0))
  w2d = pl.BlockSpec((_H, _H), lambda i: (0, 0))
  v1d = pl.BlockSpec((1, _H), lambda i: (0, 0))
  return pl.pallas_call(
      _tc1_body,
      grid=grid,
      in_specs=[tnh, tnh, cnh, w2d, w2d, v1d, v1d, v1d],
      out_specs=tnh,
      out_shape=jax.ShapeDtypeStruct((_T, _N, _H), jnp.float32),
  )(aggx, x_seq, cnt2, wlT, wrT, b, g, be)


def _tc2_body(aggh_ref, h1_ref, cnt_ref, wl_ref, wr_ref, b2_ref, g2_ref,
              be2_ref, wih_ref, whh_ref, bih_ref, bhh_ref, wa_ref, ba_ref,
              wc1_ref, bc1_ref, wc2_ref, bc2_ref, out_ref):
  cnt = jnp.maximum(cnt_ref[0, :, 0:1] + cnt_ref[1, :, 0:1], 1.0)
  hs2 = []
  for t in range(_T):
    mean = aggh_ref[t] / cnt
    res = h1_ref[t]
    lin = (jnp.dot(mean, wl_ref[...], preferred_element_type=jnp.float32, precision=lax.Precision.HIGHEST)
           + jnp.dot(res, wr_ref[...], preferred_element_type=jnp.float32, precision=lax.Precision.HIGHEST)
           + b2_ref[...])
    mu = jnp.mean(lin, axis=-1, keepdims=True)
    var = jnp.mean((lin - mu) * (lin - mu), axis=-1, keepdims=True)
    h = (lin - mu) * lax.rsqrt(var + 1e-5) * g2_ref[...] + be2_ref[...]
    hs2.append(jnp.maximum(h, 0.0) + res)

  h = jnp.zeros((_BN, _H), jnp.float32)
  outs = []
  for t in range(_T):
    xt = hs2[t]
    gi = (jnp.dot(xt, wih_ref[...], preferred_element_type=jnp.float32, precision=lax.Precision.HIGHEST)
          + bih_ref[...])
    gh = (jnp.dot(h, whh_ref[...], preferred_element_type=jnp.float32, precision=lax.Precision.HIGHEST)
          + bhh_ref[...])
    r = jax.nn.sigmoid(gi[:, 0:_H] + gh[:, 0:_H])
    zg = jax.nn.sigmoid(gi[:, _H:2 * _H] + gh[:, _H:2 * _H])
    n = jnp.tanh(gi[:, 2 * _H:3 * _H] + r * gh[:, 2 * _H:3 * _H])
    h = (1.0 - zg) * n + zg * h
    outs.append(h)

  wa = wa_ref[...]
  ba = ba_ref[0, 0]
  logit = [jnp.sum(outs[t] * wa, axis=-1, keepdims=True) + ba
           for t in range(_T)]
  m = logit[0]
  for t in range(1, _T):
    m = jnp.maximum(m, logit[t])
  e = [jnp.exp(logit[t] - m) for t in range(_T)]
  ssum = e[0]
  for t in range(1, _T):
    ssum = ssum + e[t]
  z = outs[0] * (e[0] / ssum)
  for t in range(1, _T):
    z = z + outs[t] * (e[t] / ssum)

  z1 = jnp.maximum(
      jnp.dot(z, wc1_ref[...], preferred_element_type=jnp.float32, precision=lax.Precision.HIGHEST)
      + bc1_ref[...], 0.0)
  out_ref[...] = (jnp.dot(z1, wc2_ref[...], preferred_element_type=jnp.float32, precision=lax.Precision.HIGHEST)
                  + bc2_ref[...])


def _tc2(aggh, h1, cnt2, wlT, wrT, b2, g2, be2, wihT, whhT, bih, bhh, wa, ba,
         wc1T, bc1, wc2T, bc2):
  grid = (_N // _BN,)
  tnh = pl.BlockSpec((_T, _BN, _H), lambda i: (0, i, 0))
  cnh = pl.BlockSpec((2, _BN, _H), lambda i: (0, i, 0))
  w2d = pl.BlockSpec((_H, _H), lambda i: (0, 0))
  v1d = pl.BlockSpec((1, _H), lambda i: (0, 0))
  w3h = pl.BlockSpec((_H, 3 * _H), lambda i: (0, 0))
  v3h = pl.BlockSpec((1, 3 * _H), lambda i: (0, 0))
  return pl.pallas_call(
      _tc2_body,
      grid=grid,
      in_specs=[tnh, tnh, cnh,
                w2d, w2d, v1d, v1d, v1d,
                w3h, w3h, v3h, v3h,
                v1d, pl.BlockSpec((1, 1), lambda i: (0, 0)),
                w2d, v1d, w2d, v1d],
      out_specs=pl.BlockSpec((_BN, _H), lambda i: (i, 0)),
      out_shape=jax.ShapeDtypeStruct((_N, _H), jnp.float32),
  )(aggh, h1, cnt2, wlT, wrT, b2, g2, be2, wihT, whhT, bih, bhh, wa, ba,
    wc1T, bc1, wc2T, bc2)


def kernel(x_seq, edge_index, Wl1, Wr1, b1, g1, be1, Wl2, Wr2, b2, g2, be2,
           w_ih, w_hh, b_ih, b_hh, Wa, ba, Wc1, bc1, Wc2, bc2):
  src = edge_index[0]
  dst = edge_index[1]
  x_flat = x_seq.reshape(_T * _N, _H)

  aggx, cnt2 = _make_sc_agg(True)(x_flat, src, dst)
  cnt2 = cnt2.reshape(2, _N, _H)
  h1 = _tc1(aggx.reshape(_T, _N, _H), x_seq, cnt2,
            Wl1.T, Wr1.T, b1[None], g1[None], be1[None])

  aggh = _make_sc_agg(False)(h1.reshape(_T * _N, _H), src, dst)

  wc1T = jnp.zeros((_H, _H), jnp.float32).at[:, :_H // 2].set(Wc1.T)
  bc1p = jnp.zeros((1, _H), jnp.float32).at[0, :_H // 2].set(bc1)
  wc2T = jnp.zeros((_H, _H), jnp.float32).at[:_H // 2, :2].set(Wc2.T)
  bc2p = jnp.zeros((1, _H), jnp.float32).at[0, :2].set(bc2)

  out128 = _tc2(aggh.reshape(_T, _N, _H), h1, cnt2,
                Wl2.T, Wr2.T, b2[None], g2[None], be2[None],
                w_ih.T, w_hh.T, b_ih[None], b_hh[None],
                Wa, ba.reshape(1, 1), wc1T, bc1p, wc2T, bc2p)
  return out128[:, :2]
